# Initial kernel scaffold; baseline (speedup 1.0000x reference)
import jax
import jax.numpy as jnp
from jax.experimental import pallas as pl


def kernel(X, table):
    raise NotImplementedError("write your pallas kernel here")


# speedup vs baseline: 2.5998x; 2.5998x over previous
import functools

import numpy as np
import jax
import jax.numpy as jnp
from jax import lax
from jax.experimental import pallas as pl
from jax.experimental.pallas import tpu as pltpu
from jax.experimental.pallas import tpu_sc as plsc

_VOCAB = 100000
_DIM = 64
_BATCH = 1024
_SEQ = 200
_MAX_LEN = 512

_NUM_CORES = 2
_NUM_SUBCORES = 16
_NUM_WORKERS = _NUM_CORES * _NUM_SUBCORES
_SEQ_PER_W = _BATCH // _NUM_WORKERS
_LANES = 16


def _positional_encoding_np(max_len, d):
    pos = np.arange(max_len, dtype=np.float64)[:, None]
    i = np.arange(0, d, 2, dtype=np.float64)
    angles = pos / np.power(10000.0, i / d)
    pe = np.zeros((max_len, d), dtype=np.float64)
    pe[:, 0::2] = np.sin(angles)
    pe[:, 1::2] = np.cos(angles)
    return pe.astype(np.float32)


_PE = _positional_encoding_np(_MAX_LEN, _DIM)[:_SEQ]

_mesh = plsc.VectorSubcoreMesh(
    core_axis_name="c", subcore_axis_name="s", num_cores=_NUM_CORES
)


@functools.partial(
    pl.kernel,
    out_type=jax.ShapeDtypeStruct((_BATCH, _SEQ, _DIM), jnp.float32),
    mesh=_mesh,
    compiler_params=pltpu.CompilerParams(use_tc_tiling_on_sc=False),
    scratch_types=[
        pltpu.VMEM((_SEQ,), jnp.int32),
        pltpu.VMEM((_SEQ, _DIM), jnp.float32),
        pltpu.VMEM((_SEQ, _DIM), jnp.float32),
        pltpu.SemaphoreType.DMA,
    ],
)
def _emb_kernel(x_hbm, pe_hbm, table_hbm, out_hbm, idx_v, rows_v, pe_v, sem):
    wid = lax.axis_index("s") * _NUM_CORES + lax.axis_index("c")

    pltpu.sync_copy(pe_hbm, pe_v)

    def seq_body(i, carry):
        seq = wid * _SEQ_PER_W + i
        pltpu.sync_copy(x_hbm.at[seq], idx_v)
        pltpu.async_copy(table_hbm.at[idx_v], rows_v, sem).wait()

        def add_body(r, c2):
            for j in range(_DIM // _LANES):
                sl = pl.ds(j * _LANES, _LANES)
                rows_v[r, sl] = rows_v[r, sl] + pe_v[r, sl]
            return c2

        lax.fori_loop(0, _SEQ, add_body, 0)
        pltpu.sync_copy(rows_v, out_hbm.at[seq])
        return carry

    lax.fori_loop(0, _SEQ_PER_W, seq_body, 0)


def kernel(X, table):
    pe = jnp.asarray(_PE)
    return _emb_kernel(X, pe, table)
